# baseline (speedup 1.0000x reference)
import functools

import jax
import jax.numpy as jnp
from jax import lax
from jax.experimental import pallas as pl
from jax.experimental.pallas import tpu as pltpu
from jax.experimental.pallas import tpu_sc as plsc

BATCH = 16384
HIST = 50
DIM = 64
B = BATCH * HIST
NUM_CORES = 2
NUM_SUBCORES = 16
NW = NUM_CORES * NUM_SUBCORES
B_PER_W = B // NW
CHUNK = 128
N_CHUNKS = B_PER_W // CHUNK
K = 8
D = 4


def _gather_body(idx_hbm, table_hbm, out_hbm, idx_v, *scr):
    bufs = scr[0:K]
    gsem = scr[K:2 * K]
    ssem = scr[2 * K:3 * K]

    wid = lax.axis_index("s") * NUM_CORES + lax.axis_index("c")
    base = wid * B_PER_W
    pltpu.sync_copy(idx_hbm.at[wid], idx_v)

    def fire_g(c, slot):
        pltpu.async_copy(table_hbm.at[idx_v.at[c]], bufs[slot], gsem[slot])

    def wait_g(c, slot):
        pltpu.make_async_copy(
            table_hbm.at[idx_v.at[c]], bufs[slot], gsem[slot]).wait()

    def fire_s(c, slot):
        pltpu.async_copy(
            bufs[slot], out_hbm.at[pl.ds(base + c * CHUNK, CHUNK)], ssem[slot])

    def wait_s(c, slot):
        pltpu.make_async_copy(
            bufs[slot], out_hbm.at[pl.ds(base + c * CHUNK, CHUNK)],
            ssem[slot]).wait()

    for c in range(K):
        fire_g(c, c)
        if c >= D:
            wait_g(c - D, c - D)
            fire_s(c - D, c - D)

    def outer(ii, carry):
        i = K + ii * K
        for b in range(K):
            c = i + b
            wait_s(c - K, b)
            fire_g(c, b)
            sb = (b + K - D) % K
            wait_g(c - D, sb)
            fire_s(c - D, sb)
        return carry

    lax.fori_loop(0, (N_CHUNKS - K) // K, outer, 0)

    for r in range(N_CHUNKS - D, N_CHUNKS):
        wait_g(r, r % K)
        fire_s(r, r % K)
    for r in range(N_CHUNKS - K, N_CHUNKS):
        wait_s(r, r % K)


@jax.jit
def _gather(idx, table):
    mesh = plsc.VectorSubcoreMesh(core_axis_name="c", subcore_axis_name="s")
    scratch = [pltpu.VMEM((N_CHUNKS, CHUNK), jnp.int32)]
    scratch += [pltpu.VMEM((CHUNK, DIM), jnp.float32) for _ in range(K)]
    scratch += [pltpu.SemaphoreType.DMA for _ in range(2 * K)]
    kfn = functools.partial(
        pl.kernel,
        mesh=mesh,
        out_type=jax.ShapeDtypeStruct((B, DIM), jnp.float32),
        scratch_types=scratch,
        compiler_params=pltpu.CompilerParams(use_tc_tiling_on_sc=False),
    )(_gather_body)
    return kfn(idx, table)


NB = 128


def _relayout_body(src_ref, dst_ref):
    dst_ref[...] = src_ref[...].reshape(dst_ref.shape)


@jax.jit
def _relayout(flat):
    return pl.pallas_call(
        _relayout_body,
        grid=(BATCH // NB,),
        in_specs=[pl.BlockSpec((NB * HIST, DIM), lambda i: (i, 0))],
        out_specs=pl.BlockSpec((NB, HIST, DIM), lambda i: (i, 0, 0)),
        out_shape=jax.ShapeDtypeStruct((BATCH, HIST, DIM), jnp.float32),
    )(flat)


NUM_USERS = 100000
UROWS = 1000


def _copy_body(src_ref, dst_ref):
    dst_ref[...] = src_ref[...]


@jax.jit
def _user_copy(user_table):
    return pl.pallas_call(
        _copy_body,
        grid=(NUM_USERS // UROWS,),
        in_specs=[pl.BlockSpec((UROWS, DIM), lambda i: (i, 0))],
        out_specs=pl.BlockSpec((UROWS, DIM), lambda i: (i, 0)),
        out_shape=jax.ShapeDtypeStruct((NUM_USERS, DIM), jnp.float32),
    )(user_table)


def _smooth_body(time_ref, bw_ref, out_ref):
    t = lax.broadcasted_iota(jnp.int32, (24, 24), 1)
    tn = lax.broadcasted_iota(jnp.int32, (24, 24), 0)
    d = jnp.abs(t - tn).astype(jnp.float32)
    dist = jnp.minimum(d, 24.0 - d)
    z = dist / bw_ref[0]
    w = jnp.exp(-0.5 * z * z)
    out_ref[...] = jnp.dot(w, time_ref[...], preferred_element_type=jnp.float32)


@jax.jit
def _smooth(time_table, bandwidth):
    return pl.pallas_call(
        _smooth_body,
        out_shape=jax.ShapeDtypeStruct((24, DIM), jnp.float32),
        in_specs=[
            pl.BlockSpec(memory_space=pltpu.VMEM),
            pl.BlockSpec(memory_space=pltpu.SMEM),
        ],
        out_specs=pl.BlockSpec(memory_space=pltpu.VMEM),
    )(time_table, bandwidth)


def kernel(location_x, loc_table, user_table, time_table, bandwidth):
    idx = location_x.reshape(NW, N_CHUNKS, CHUNK)
    loc_flat = _gather(idx, loc_table)
    loc_embedded = _relayout(loc_flat)
    smoothed = _smooth(time_table, bandwidth)
    return (loc_embedded, time_table, smoothed, _user_copy(user_table))


# speedup vs baseline: 1.2973x; 1.2973x over previous
import functools

import jax
import jax.numpy as jnp
from jax import lax
from jax.experimental import pallas as pl
from jax.experimental.pallas import tpu as pltpu
from jax.experimental.pallas import tpu_sc as plsc

BATCH = 16384
HIST = 50
HIST_PAD = 56
DIM = 64
IDXPAD = 64
B_PAD = BATCH * HIST_PAD
NUM_CORES = 2
NUM_SUBCORES = 16
NW = NUM_CORES * NUM_SUBCORES
BATCH_PER_W = BATCH // NW
K = 8
D = 4


def _gather_body(idx_hbm, table_hbm, out_hbm, idx_v, *scr):
    bufs = scr[0:K]
    gsem = scr[K:2 * K]
    ssem = scr[2 * K:3 * K]

    wid = lax.axis_index("s") * NUM_CORES + lax.axis_index("c")
    base_b = wid * BATCH_PER_W
    pltpu.sync_copy(idx_hbm.at[pl.ds(base_b, BATCH_PER_W)], idx_v)

    def fire_g(c, slot):
        pltpu.async_copy(
            table_hbm.at[idx_v.at[c, pl.ds(0, HIST_PAD)]], bufs[slot],
            gsem[slot])

    def wait_g(c, slot):
        pltpu.make_async_copy(
            table_hbm.at[idx_v.at[c, pl.ds(0, HIST_PAD)]], bufs[slot],
            gsem[slot]).wait()

    def fire_s(c, slot):
        pltpu.async_copy(
            bufs[slot], out_hbm.at[pl.ds((base_b + c) * HIST_PAD, HIST_PAD)],
            ssem[slot])

    def wait_s(c, slot):
        pltpu.make_async_copy(
            bufs[slot], out_hbm.at[pl.ds((base_b + c) * HIST_PAD, HIST_PAD)],
            ssem[slot]).wait()

    for c in range(K):
        fire_g(c, c)
        if c >= D:
            wait_g(c - D, c - D)
            fire_s(c - D, c - D)

    def outer(ii, carry):
        i = K + ii * K
        for b in range(K):
            c = i + b
            wait_s(c - K, b)
            fire_g(c, b)
            sb = (b + K - D) % K
            wait_g(c - D, sb)
            fire_s(c - D, sb)
        return carry

    lax.fori_loop(0, (BATCH_PER_W - K) // K, outer, 0)

    for r in range(BATCH_PER_W - D, BATCH_PER_W):
        wait_g(r, r % K)
        fire_s(r, r % K)
    for r in range(BATCH_PER_W - K, BATCH_PER_W):
        wait_s(r, r % K)


@jax.jit
def _gather(idx, table):
    mesh = plsc.VectorSubcoreMesh(core_axis_name="c", subcore_axis_name="s")
    scratch = [pltpu.VMEM((BATCH_PER_W, IDXPAD), jnp.int32)]
    scratch += [pltpu.VMEM((HIST_PAD, DIM), jnp.float32) for _ in range(K)]
    scratch += [pltpu.SemaphoreType.DMA for _ in range(2 * K)]
    kfn = functools.partial(
        pl.kernel,
        mesh=mesh,
        out_type=jax.ShapeDtypeStruct((B_PAD, DIM), jnp.float32),
        scratch_types=scratch,
        compiler_params=pltpu.CompilerParams(use_tc_tiling_on_sc=False),
    )(_gather_body)
    return kfn(idx, table)


NUM_USERS = 100000
UROWS = 1000


def _copy_body(src_ref, dst_ref):
    dst_ref[...] = src_ref[...]


@jax.jit
def _user_copy(user_table):
    return pl.pallas_call(
        _copy_body,
        grid=(NUM_USERS // UROWS,),
        in_specs=[pl.BlockSpec((UROWS, DIM), lambda i: (i, 0))],
        out_specs=pl.BlockSpec((UROWS, DIM), lambda i: (i, 0)),
        out_shape=jax.ShapeDtypeStruct((NUM_USERS, DIM), jnp.float32),
    )(user_table)


def _smooth_body(time_ref, bw_ref, out_ref):
    t = lax.broadcasted_iota(jnp.int32, (24, 24), 1)
    tn = lax.broadcasted_iota(jnp.int32, (24, 24), 0)
    d = jnp.abs(t - tn).astype(jnp.float32)
    dist = jnp.minimum(d, 24.0 - d)
    z = dist / bw_ref[0]
    w = jnp.exp(-0.5 * z * z)
    out_ref[...] = jnp.dot(w, time_ref[...], preferred_element_type=jnp.float32)


@jax.jit
def _smooth(time_table, bandwidth):
    return pl.pallas_call(
        _smooth_body,
        out_shape=jax.ShapeDtypeStruct((24, DIM), jnp.float32),
        in_specs=[
            pl.BlockSpec(memory_space=pltpu.VMEM),
            pl.BlockSpec(memory_space=pltpu.SMEM),
        ],
        out_specs=pl.BlockSpec(memory_space=pltpu.VMEM),
    )(time_table, bandwidth)


NUM_LOCS = 1000000


def kernel(location_x, loc_table, user_table, time_table, bandwidth):
    npad = IDXPAD - HIST
    fill = (jnp.arange(BATCH, dtype=jnp.int32)[:, None] * npad
            + jnp.arange(npad, dtype=jnp.int32)[None, :]) % NUM_LOCS
    idx = jnp.concatenate([location_x.astype(jnp.int32), fill], axis=1)
    loc_flat = _gather(idx, loc_table)
    loc_embedded = loc_flat.reshape(BATCH, HIST_PAD, DIM)[:, :HIST, :]
    smoothed = _smooth(time_table, bandwidth)
    return (loc_embedded, time_table, smoothed, _user_copy(user_table))


# speedup vs baseline: 1.3668x; 1.0536x over previous
import functools

import jax
import jax.numpy as jnp
from jax import lax
from jax.experimental import pallas as pl
from jax.experimental.pallas import tpu as pltpu
from jax.experimental.pallas import tpu_sc as plsc

BATCH = 16384
HIST = 50
DIM = 64
B = BATCH * HIST
NUM_CORES = 2
NUM_SUBCORES = 16
NW = NUM_CORES * NUM_SUBCORES
B_PER_W = B // NW
CHUNK = 128
N_CHUNKS = B_PER_W // CHUNK
K = 4
D = 2


def _gather_body(idx_hbm, table_hbm, out_hbm, idx_v, *scr):
    bufs = scr[0:K]
    gsem = scr[K:2 * K]
    ssem = scr[2 * K:3 * K]

    wid = lax.axis_index("s") * NUM_CORES + lax.axis_index("c")
    base = wid * B_PER_W
    pltpu.sync_copy(idx_hbm.at[wid], idx_v)

    def fire_g(c, slot):
        pltpu.async_copy(table_hbm.at[idx_v.at[c]], bufs[slot], gsem[slot])

    def wait_g(c, slot):
        pltpu.make_async_copy(
            table_hbm.at[idx_v.at[c]], bufs[slot], gsem[slot]).wait()

    def fire_s(c, slot):
        pltpu.async_copy(
            bufs[slot].at[pl.ds(0, CHUNK), pl.ds(0, DIM)],
            out_hbm.at[pl.ds(base + c * CHUNK, CHUNK)], ssem[slot])

    def wait_s(c, slot):
        pltpu.make_async_copy(
            bufs[slot].at[pl.ds(0, CHUNK), pl.ds(0, DIM)],
            out_hbm.at[pl.ds(base + c * CHUNK, CHUNK)],
            ssem[slot]).wait()

    for c in range(K):
        fire_g(c, c)
        if c >= D:
            wait_g(c - D, c - D)
            fire_s(c - D, c - D)

    def outer(ii, carry):
        i = K + ii * K
        for b in range(K):
            c = i + b
            wait_s(c - K, b)
            fire_g(c, b)
            sb = (b + K - D) % K
            wait_g(c - D, sb)
            fire_s(c - D, sb)
        return carry

    lax.fori_loop(0, (N_CHUNKS - K) // K, outer, 0)

    for r in range(N_CHUNKS - D, N_CHUNKS):
        wait_g(r, r % K)
        fire_s(r, r % K)
    for r in range(N_CHUNKS - K, N_CHUNKS):
        wait_s(r, r % K)


@jax.jit
def _gather(idx, table):
    mesh = plsc.VectorSubcoreMesh(core_axis_name="c", subcore_axis_name="s")
    scratch = [pltpu.VMEM((N_CHUNKS, CHUNK), jnp.int32)]
    scratch += [pltpu.VMEM((CHUNK, 128), jnp.float32) for _ in range(K)]
    scratch += [pltpu.SemaphoreType.DMA for _ in range(2 * K)]
    kfn = functools.partial(
        pl.kernel,
        mesh=mesh,
        out_type=jax.ShapeDtypeStruct((B, DIM), jnp.float32),
        scratch_types=scratch,
        compiler_params=pltpu.CompilerParams(use_tc_tiling_on_sc=False),
    )(_gather_body)
    return kfn(idx, table)


NUM_USERS = 100000
UROWS = 1000


def _copy_body(src_ref, dst_ref):
    dst_ref[...] = src_ref[...]


@jax.jit
def _user_copy(user_table):
    return pl.pallas_call(
        _copy_body,
        grid=(NUM_USERS // UROWS,),
        in_specs=[pl.BlockSpec((UROWS, DIM), lambda i: (i, 0))],
        out_specs=pl.BlockSpec((UROWS, DIM), lambda i: (i, 0)),
        out_shape=jax.ShapeDtypeStruct((NUM_USERS, DIM), jnp.float32),
    )(user_table)


def _smooth_body(time_ref, bw_ref, out_ref):
    t = lax.broadcasted_iota(jnp.int32, (24, 24), 1)
    tn = lax.broadcasted_iota(jnp.int32, (24, 24), 0)
    d = jnp.abs(t - tn).astype(jnp.float32)
    dist = jnp.minimum(d, 24.0 - d)
    z = dist / bw_ref[0]
    w = jnp.exp(-0.5 * z * z)
    out_ref[...] = jnp.dot(w, time_ref[...], preferred_element_type=jnp.float32)


@jax.jit
def _smooth(time_table, bandwidth):
    return pl.pallas_call(
        _smooth_body,
        out_shape=jax.ShapeDtypeStruct((24, DIM), jnp.float32),
        in_specs=[
            pl.BlockSpec(memory_space=pltpu.VMEM),
            pl.BlockSpec(memory_space=pltpu.SMEM),
        ],
        out_specs=pl.BlockSpec(memory_space=pltpu.VMEM),
    )(time_table, bandwidth)


def kernel(location_x, loc_table, user_table, time_table, bandwidth):
    idx = location_x.reshape(NW, N_CHUNKS, CHUNK)
    table128 = jnp.pad(loc_table, ((0, 0), (0, 128 - DIM)))
    loc_flat = _gather(idx, table128)
    loc_embedded = loc_flat.reshape(BATCH, HIST, DIM)
    smoothed = _smooth(time_table, bandwidth)
    return (loc_embedded, time_table, smoothed, user_table)


# speedup vs baseline: 1.3762x; 1.0069x over previous
import functools

import jax
import jax.numpy as jnp
from jax import lax
from jax.experimental import pallas as pl
from jax.experimental.pallas import tpu as pltpu
from jax.experimental.pallas import tpu_sc as plsc

BATCH = 16384
HIST = 50
DIM = 64
B = BATCH * HIST
NUM_CORES = 2
NUM_SUBCORES = 16
NW = NUM_CORES * NUM_SUBCORES
B_PER_W = B // NW
CHUNK = 128
N_CHUNKS = B_PER_W // CHUNK
K = 8
D = 4


def _gather_body(idx_hbm, table_hbm, out_hbm, idx_v, *scr):
    bufs = scr[0:K]
    gsem = scr[K:2 * K]
    ssem = scr[2 * K:3 * K]

    wid = lax.axis_index("s") * NUM_CORES + lax.axis_index("c")
    base = wid * B_PER_W
    pltpu.sync_copy(idx_hbm.at[wid], idx_v)

    def fire_g(c, slot):
        pltpu.async_copy(table_hbm.at[idx_v.at[c]], bufs[slot], gsem[slot])

    def wait_g(c, slot):
        pltpu.make_async_copy(
            table_hbm.at[idx_v.at[c]], bufs[slot], gsem[slot]).wait()

    def fire_s(c, slot):
        pltpu.async_copy(
            bufs[slot], out_hbm.at[pl.ds(base + c * CHUNK, CHUNK)], ssem[slot])

    def wait_s(c, slot):
        pltpu.make_async_copy(
            bufs[slot], out_hbm.at[pl.ds(base + c * CHUNK, CHUNK)],
            ssem[slot]).wait()

    for c in range(K):
        fire_g(c, c)
        if c >= D:
            wait_g(c - D, c - D)
            fire_s(c - D, c - D)

    def outer(ii, carry):
        i = K + ii * K
        for b in range(K):
            c = i + b
            wait_s(c - K, b)
            fire_g(c, b)
            sb = (b + K - D) % K
            wait_g(c - D, sb)
            fire_s(c - D, sb)
        return carry

    lax.fori_loop(0, (N_CHUNKS - K) // K, outer, 0)

    for r in range(N_CHUNKS - D, N_CHUNKS):
        wait_g(r, r % K)
        fire_s(r, r % K)
    for r in range(N_CHUNKS - K, N_CHUNKS):
        wait_s(r, r % K)


@jax.jit
def _gather(idx, table):
    mesh = plsc.VectorSubcoreMesh(core_axis_name="c", subcore_axis_name="s")
    scratch = [pltpu.VMEM((N_CHUNKS, CHUNK), jnp.int32)]
    scratch += [pltpu.VMEM((CHUNK, DIM), jnp.float32) for _ in range(K)]
    scratch += [pltpu.SemaphoreType.DMA for _ in range(2 * K)]
    kfn = functools.partial(
        pl.kernel,
        mesh=mesh,
        out_type=jax.ShapeDtypeStruct((B, DIM), jnp.float32),
        scratch_types=scratch,
        compiler_params=pltpu.CompilerParams(use_tc_tiling_on_sc=False),
    )(_gather_body)
    return kfn(idx, table)


def _smooth_body(time_ref, bw_ref, out_ref):
    t = lax.broadcasted_iota(jnp.int32, (24, 24), 1)
    tn = lax.broadcasted_iota(jnp.int32, (24, 24), 0)
    d = jnp.abs(t - tn).astype(jnp.float32)
    dist = jnp.minimum(d, 24.0 - d)
    z = dist / bw_ref[0]
    w = jnp.exp(-0.5 * z * z)
    out_ref[...] = jnp.dot(w, time_ref[...], preferred_element_type=jnp.float32)


@jax.jit
def _smooth(time_table, bandwidth):
    return pl.pallas_call(
        _smooth_body,
        out_shape=jax.ShapeDtypeStruct((24, DIM), jnp.float32),
        in_specs=[
            pl.BlockSpec(memory_space=pltpu.VMEM),
            pl.BlockSpec(memory_space=pltpu.SMEM),
        ],
        out_specs=pl.BlockSpec(memory_space=pltpu.VMEM),
    )(time_table, bandwidth)


def kernel(location_x, loc_table, user_table, time_table, bandwidth):
    idx = location_x.reshape(NW, N_CHUNKS, CHUNK)
    loc_flat = _gather(idx, loc_table)
    loc_embedded = loc_flat.reshape(BATCH, HIST, DIM)
    smoothed = _smooth(time_table, bandwidth)
    return (loc_embedded, time_table, smoothed, user_table)
